# Initial kernel scaffold; baseline (speedup 1.0000x reference)
import jax
import jax.numpy as jnp
from jax.experimental import pallas as pl


def kernel(y_pred, y_attack):
    raise NotImplementedError("write your pallas kernel here")


# speedup vs baseline: 1.4365x; 1.4365x over previous
import functools

import jax
import jax.numpy as jnp
from jax import lax
from jax.experimental import pallas as pl
from jax.experimental.pallas import tpu as pltpu
from jax.experimental.pallas import tpu_sc as plsc

_B = 1024
_C = 100000
_K = 8
_BLK = 2048
_NBLK = (_C + _BLK - 1) // _BLK
_NCHUNK = _BLK // 128
_RB = 256
_NRB = _B // _RB
_NEG_INF = float("-inf")

_SC_CORES = 2
_SC_SUBCORES = 16
_NW = _SC_CORES * _SC_SUBCORES
_IDX_PER_W = (_B * _K) // _NW
_IDX_ROWS = _IDX_PER_W // 128


def _sc_gather_kernel(idx_hbm, src_hbm, out_hbm, idx_v, vals_v, sem):
    wid = lax.axis_index("s") * _SC_CORES + lax.axis_index("c")
    pltpu.sync_copy(idx_hbm.at[wid], idx_v)
    for r in range(_IDX_ROWS):
        ri = jnp.int32(r)
        pltpu.async_copy(src_hbm.at[idx_v.at[ri]], vals_v.at[ri], sem).wait()
    pltpu.sync_copy(vals_v, out_hbm.at[wid])


def _gather_attack(y_flat, flat_idx):
    mesh = plsc.VectorSubcoreMesh(core_axis_name="c", subcore_axis_name="s")
    run = pl.kernel(
        _sc_gather_kernel,
        out_type=jax.ShapeDtypeStruct((_NW, _IDX_ROWS, 128), jnp.float32),
        mesh=mesh,
        scratch_types=[
            pltpu.VMEM((_IDX_ROWS, 128), jnp.int32),
            pltpu.VMEM((_IDX_ROWS, 128), jnp.float32),
            pltpu.SemaphoreType.DMA,
        ],
    )
    return run(flat_idx, y_flat)


def _scan_kernel(flags_ref, att_ref, x_ref, m_ref, s_ref, mm_ref):
    j = pl.program_id(1)
    lane = lax.broadcasted_iota(jnp.int32, (_RB, 128), 1)

    @pl.when(j == 0)
    def _init():
        m_ref[...] = jnp.full((_RB, 128), _NEG_INF, jnp.float32)
        s_ref[...] = jnp.zeros((_RB, 128), jnp.float32)
        mm_ref[...] = jnp.full((_RB, 128), _NEG_INF, jnp.float32)

    base = j * _BLK
    last = _NBLK - 1

    def run(tail, attack):
        def chunk(g):
            xg = x_ref[:, g * 128:(g + 1) * 128]
            if tail:
                rem = _C - (base + g * 128)
                xg = jnp.where(lane < rem, xg, _NEG_INF)
            return xg

        bm = chunk(0)
        for g in range(1, _NCHUNK):
            bm = jnp.maximum(bm, chunk(g))

        m_old = m_ref[...]
        m_new = jnp.maximum(m_old, bm)

        bs = jnp.exp(chunk(0) - m_new)
        for g in range(1, _NCHUNK):
            bs = bs + jnp.exp(chunk(g) - m_new)
        s_ref[...] = s_ref[...] * jnp.exp(m_old - m_new) + bs
        m_ref[...] = m_new

        if attack:
            a_rel = att_ref[...] - base
            bks = [jnp.broadcast_to(a_rel[:, k:k + 1], (_RB, 128))
                   for k in range(_K)]
            mmx = mm_ref[...]
            for g in range(_NCHUNK):
                lg = lane + g * 128
                hit = lg == bks[0]
                for k in range(1, _K):
                    hit = hit | (lg == bks[k])
                mmx = jnp.maximum(mmx, jnp.where(hit, _NEG_INF, chunk(g)))
            mm_ref[...] = mmx
        else:
            mm_ref[...] = jnp.maximum(mm_ref[...], bm)

    @pl.when(j == last)
    def _tail():
        run(True, True)

    @pl.when((j < last) & (flags_ref[j] != 0))
    def _masked():
        run(False, True)

    @pl.when((j < last) & (flags_ref[j] == 0))
    def _plain():
        run(False, False)


def _combine_kernel(att_ref, m_ref, s_ref, mm_ref, out_ref):
    m_l = m_ref[...]
    m = jnp.max(m_l, axis=1, keepdims=True)
    s = jnp.sum(s_ref[...] * jnp.exp(m_l - m), axis=1, keepdims=True)
    mm = jnp.max(mm_ref[...], axis=1, keepdims=True)

    p = jnp.exp(att_ref[...] - m) / s
    p_mm = jnp.exp(mm - m) / s
    macro = p_mm - jnp.min(p, axis=1, keepdims=True)

    d = p[:, 1:] - p[:, :-1]
    d = ((5.0 + 5.0 * d) - 5.0) * 0.2
    w = jnp.exp(9.0 * jnp.log(1.0 + d)) - 1.0
    wm = jnp.sum(w, axis=1, keepdims=True) * (1.0 / (_K - 1))
    sl = jnp.exp(jnp.log(1.0 + wm) / 9.0) - 1.0

    cm = ((5.0 + 5.0 * macro) - 5.0) * 0.2
    cs = ((5.0 + 5.0 * sl) - 5.0) * 0.2
    v = 0.5 * (jnp.exp(10.0 * jnp.log(1.0 + cm))
               + jnp.exp(10.0 * jnp.log(1.0 + cs))) - 1.0
    out_ref[...] = jnp.exp(jnp.log(1.0 + v) / 10.0) - 1.0


def _z(*_args):
    return jnp.int32(0)


def _finish(y_pred, att32, att_vals, flags):
    grid_spec = pltpu.PrefetchScalarGridSpec(
        num_scalar_prefetch=1,
        grid=(_NRB, _NBLK),
        in_specs=[
            pl.BlockSpec((_RB, _K), lambda i, j, flags: (i, _z())),
            pl.BlockSpec((_RB, _BLK), lambda i, j, flags: (i, j)),
        ],
        out_specs=[
            pl.BlockSpec((_RB, 128), lambda i, j, flags: (i, _z())),
            pl.BlockSpec((_RB, 128), lambda i, j, flags: (i, _z())),
            pl.BlockSpec((_RB, 128), lambda i, j, flags: (i, _z())),
        ],
    )
    m_l, s_l, mm_l = pl.pallas_call(
        _scan_kernel,
        grid_spec=grid_spec,
        out_shape=[jax.ShapeDtypeStruct((_B, 128), jnp.float32)] * 3,
        compiler_params=pltpu.CompilerParams(
            dimension_semantics=("arbitrary", "arbitrary"),
        ),
    )(flags, att32, y_pred)

    out = pl.pallas_call(
        _combine_kernel,
        out_shape=jax.ShapeDtypeStruct((_B, 1), jnp.float32),
    )(att_vals, m_l, s_l, mm_l)
    return out.reshape(_B)


@jax.jit
def kernel(y_pred, y_attack):
    att32 = y_attack.astype(jnp.int32)
    flags = (
        jnp.zeros((_NBLK,), jnp.int32)
        .at[(att32 // _BLK).reshape(-1)]
        .set(1, mode="drop")
    )
    rows = jnp.arange(_B, dtype=jnp.int32)[:, None]
    flat_idx = (rows * _C + att32).reshape(_NW, _IDX_ROWS, 128)
    att_vals = _gather_attack(y_pred.reshape(-1), flat_idx)
    att_vals = att_vals.reshape(_B, _K)
    return _finish(y_pred, att32, att_vals, flags)


# speedup vs baseline: 1.4969x; 1.0421x over previous
import functools

import jax
import jax.numpy as jnp
from jax import lax
from jax.experimental import pallas as pl
from jax.experimental.pallas import tpu as pltpu
from jax.experimental.pallas import tpu_sc as plsc

_B = 1024
_C = 100000
_K = 8
_BLK = 2048
_NBLK = (_C + _BLK - 1) // _BLK
_NCHUNK = _BLK // 128
_RB = 1024
_NRB = _B // _RB
_NEG_INF = float("-inf")

_SC_CORES = 2
_SC_SUBCORES = 16
_NW = _SC_CORES * _SC_SUBCORES
_IDX_PER_W = (_B * _K) // _NW
_IDX_ROWS = _IDX_PER_W // 128


def _sc_gather_kernel(idx_hbm, src_hbm, out_hbm, idx_v, vals_v, sem):
    wid = lax.axis_index("s") * _SC_CORES + lax.axis_index("c")
    pltpu.sync_copy(idx_hbm.at[wid], idx_v)
    for r in range(_IDX_ROWS):
        ri = jnp.int32(r)
        pltpu.async_copy(src_hbm.at[idx_v.at[ri]], vals_v.at[ri], sem).wait()
    pltpu.sync_copy(vals_v, out_hbm.at[wid])


def _gather_attack(y_flat, flat_idx):
    mesh = plsc.VectorSubcoreMesh(core_axis_name="c", subcore_axis_name="s")
    run = pl.kernel(
        _sc_gather_kernel,
        out_type=jax.ShapeDtypeStruct((_NW, _IDX_ROWS, 128), jnp.float32),
        mesh=mesh,
        scratch_types=[
            pltpu.VMEM((_IDX_ROWS, 128), jnp.int32),
            pltpu.VMEM((_IDX_ROWS, 128), jnp.float32),
            pltpu.SemaphoreType.DMA,
        ],
    )
    return run(flat_idx, y_flat)


def _scan_kernel(flags_ref, att_ref, x_ref, m_ref, s_ref, mm_ref):
    j = pl.program_id(1)
    lane = lax.broadcasted_iota(jnp.int32, (_RB, 128), 1)

    @pl.when(j == 0)
    def _init():
        m_ref[...] = jnp.full((_RB, 128), _NEG_INF, jnp.float32)
        s_ref[...] = jnp.zeros((_RB, 128), jnp.float32)
        mm_ref[...] = jnp.full((_RB, 128), _NEG_INF, jnp.float32)

    base = j * _BLK
    last = _NBLK - 1

    def run(tail, attack):
        def chunk(g):
            xg = x_ref[:, g * 128:(g + 1) * 128]
            if tail:
                rem = _C - (base + g * 128)
                xg = jnp.where(lane < rem, xg, _NEG_INF)
            return xg

        bm = chunk(0)
        for g in range(1, _NCHUNK):
            bm = jnp.maximum(bm, chunk(g))

        m_old = m_ref[...]
        m_new = jnp.maximum(m_old, bm)

        bs = jnp.exp(chunk(0) - m_new)
        for g in range(1, _NCHUNK):
            bs = bs + jnp.exp(chunk(g) - m_new)
        s_ref[...] = s_ref[...] * jnp.exp(m_old - m_new) + bs
        m_ref[...] = m_new

        if attack:
            a_rel = att_ref[...] - base
            bks = [jnp.broadcast_to(a_rel[:, k:k + 1], (_RB, 128))
                   for k in range(_K)]
            mmx = mm_ref[...]
            for g in range(_NCHUNK):
                lg = lane + g * 128
                hit = lg == bks[0]
                for k in range(1, _K):
                    hit = hit | (lg == bks[k])
                mmx = jnp.maximum(mmx, jnp.where(hit, _NEG_INF, chunk(g)))
            mm_ref[...] = mmx
        else:
            mm_ref[...] = jnp.maximum(mm_ref[...], bm)

    @pl.when(j == last)
    def _tail():
        run(True, True)

    @pl.when((j < last) & (flags_ref[j] != 0))
    def _masked():
        run(False, True)

    @pl.when((j < last) & (flags_ref[j] == 0))
    def _plain():
        run(False, False)


def _combine_kernel(att_ref, m_ref, s_ref, mm_ref, out_ref):
    m_l = m_ref[...]
    m = jnp.max(m_l, axis=1, keepdims=True)
    s = jnp.sum(s_ref[...] * jnp.exp(m_l - m), axis=1, keepdims=True)
    mm = jnp.max(mm_ref[...], axis=1, keepdims=True)

    p = jnp.exp(att_ref[...] - m) / s
    p_mm = jnp.exp(mm - m) / s
    macro = p_mm - jnp.min(p, axis=1, keepdims=True)

    d = p[:, 1:] - p[:, :-1]
    d = ((5.0 + 5.0 * d) - 5.0) * 0.2
    w = jnp.exp(9.0 * jnp.log(1.0 + d)) - 1.0
    wm = jnp.sum(w, axis=1, keepdims=True) * (1.0 / (_K - 1))
    sl = jnp.exp(jnp.log(1.0 + wm) / 9.0) - 1.0

    cm = ((5.0 + 5.0 * macro) - 5.0) * 0.2
    cs = ((5.0 + 5.0 * sl) - 5.0) * 0.2
    v = 0.5 * (jnp.exp(10.0 * jnp.log(1.0 + cm))
               + jnp.exp(10.0 * jnp.log(1.0 + cs))) - 1.0
    out_ref[...] = jnp.exp(jnp.log(1.0 + v) / 10.0) - 1.0


def _z(*_args):
    return jnp.int32(0)


def _finish(y_pred, att32, att_vals, flags):
    grid_spec = pltpu.PrefetchScalarGridSpec(
        num_scalar_prefetch=1,
        grid=(_NRB, _NBLK),
        in_specs=[
            pl.BlockSpec((_RB, _K), lambda i, j, flags: (i, _z())),
            pl.BlockSpec((_RB, _BLK), lambda i, j, flags: (i, j)),
        ],
        out_specs=[
            pl.BlockSpec((_RB, 128), lambda i, j, flags: (i, _z())),
            pl.BlockSpec((_RB, 128), lambda i, j, flags: (i, _z())),
            pl.BlockSpec((_RB, 128), lambda i, j, flags: (i, _z())),
        ],
    )
    m_l, s_l, mm_l = pl.pallas_call(
        _scan_kernel,
        grid_spec=grid_spec,
        out_shape=[jax.ShapeDtypeStruct((_B, 128), jnp.float32)] * 3,
        compiler_params=pltpu.CompilerParams(
            dimension_semantics=("arbitrary", "arbitrary"),
        ),
    )(flags, att32, y_pred)

    out = pl.pallas_call(
        _combine_kernel,
        out_shape=jax.ShapeDtypeStruct((_B, 1), jnp.float32),
    )(att_vals, m_l, s_l, mm_l)
    return out.reshape(_B)


@jax.jit
def kernel(y_pred, y_attack):
    att32 = y_attack.astype(jnp.int32)
    flags = (
        jnp.zeros((_NBLK,), jnp.int32)
        .at[(att32 // _BLK).reshape(-1)]
        .set(1, mode="drop")
    )
    rows = jnp.arange(_B, dtype=jnp.int32)[:, None]
    flat_idx = (rows * _C + att32).reshape(_NW, _IDX_ROWS, 128)
    att_vals = _gather_attack(y_pred.reshape(-1), flat_idx)
    att_vals = att_vals.reshape(_B, _K)
    return _finish(y_pred, att32, att_vals, flags)


# speedup vs baseline: 1.5108x; 1.0092x over previous
import jax
import jax.numpy as jnp
from jax import lax
from jax.experimental import pallas as pl
from jax.experimental.pallas import tpu as pltpu
from jax.experimental.pallas import tpu_sc as plsc

_B = 1024
_C = 100000
_K = 8
_BLK = 2048
_NBLK = (_C + _BLK - 1) // _BLK
_NCHUNK = _BLK // 128
_RB = 32
_NRB = _B // _RB
_MAIN_GROUPS = 48
_TAIL_BASE = _MAIN_GROUPS * _BLK
_TAIL_CH = 13
_XT_BASE = _TAIL_BASE + _TAIL_CH * 128
_XT_W = _C - _XT_BASE
_NEG_INF = float("-inf")

_SC_CORES = 2
_SC_SUBCORES = 16
_NW = _SC_CORES * _SC_SUBCORES
_IDX_PER_W = (_B * _K) // _NW
_IDX_ROWS = _IDX_PER_W // 128


def _sc_gather_kernel(idx_hbm, src_hbm, out_hbm, idx_v, vals_v, sem):
    wid = lax.axis_index("s") * _SC_CORES + lax.axis_index("c")
    pltpu.sync_copy(idx_hbm.at[wid], idx_v)
    for r in range(_IDX_ROWS):
        ri = jnp.int32(r)
        pltpu.async_copy(src_hbm.at[idx_v.at[ri]], vals_v.at[ri], sem).wait()
    pltpu.sync_copy(vals_v, out_hbm.at[wid])


def _gather_attack(y_flat, flat_idx):
    mesh = plsc.VectorSubcoreMesh(core_axis_name="c", subcore_axis_name="s")
    run = pl.kernel(
        _sc_gather_kernel,
        out_type=jax.ShapeDtypeStruct((_NW, _IDX_ROWS, 128), jnp.float32),
        mesh=mesh,
        scratch_types=[
            pltpu.VMEM((_IDX_ROWS, 128), jnp.int32),
            pltpu.VMEM((_IDX_ROWS, 128), jnp.float32),
            pltpu.SemaphoreType.DMA,
        ],
    )
    return run(flat_idx, y_flat)


def _scan_kernel(flags_ref, att_ref, x_ref, xt_ref, m_ref, s_ref, mm_ref):
    lane = lax.broadcasted_iota(jnp.int32, (_RB, 128), 1)

    def chunk(base, g):
        start = jnp.asarray(base, jnp.int32) + jnp.int32(g * 128)
        return x_ref[:, pl.ds(pl.multiple_of(start, 128), 128)]

    xt = jnp.concatenate(
        [xt_ref[...], jnp.full((_RB, 128 - _XT_W), _NEG_INF, jnp.float32)],
        axis=1,
    )

    def body_a(jj, m):
        base = jj * jnp.int32(_BLK)
        for g in range(_NCHUNK):
            m = jnp.maximum(m, chunk(base, g))
        return m

    m = lax.fori_loop(jnp.int32(0), jnp.int32(_MAIN_GROUPS), body_a,
                      jnp.full((_RB, 128), _NEG_INF, jnp.float32))
    for g in range(_TAIL_CH):
        m = jnp.maximum(m, chunk(_TAIL_BASE, g))
    m = jnp.maximum(m, xt)

    def group_b(jj_base, getters, s, mm, flag):
        def masked():
            a_rel = att_ref[...] - jj_base
            bks = [jnp.broadcast_to(a_rel[:, k:k + 1], (_RB, 128))
                   for k in range(_K)]
            mmx = mm
            for g, get in enumerate(getters):
                lg = lane + g * 128
                hit = lg == bks[0]
                for k in range(1, _K):
                    hit = hit | (lg == bks[k])
                mmx = jnp.maximum(mmx, jnp.where(hit, _NEG_INF, get()))
            return mmx

        def plain():
            mmp = mm
            for get in getters:
                mmp = jnp.maximum(mmp, get())
            return mmp

        for get in getters:
            s = s + jnp.exp(get() - m)
        mm = lax.cond(flag != 0, masked, plain)
        return s, mm

    def body_b(jj, carry):
        s, mm = carry
        base = jj * jnp.int32(_BLK)
        getters = [(lambda g=g: chunk(base, g)) for g in range(_NCHUNK)]
        return group_b(base, getters, s, mm, flags_ref[jj])

    s, mm = lax.fori_loop(
        jnp.int32(0), jnp.int32(_MAIN_GROUPS), body_b,
        (jnp.zeros((_RB, 128), jnp.float32),
         jnp.full((_RB, 128), _NEG_INF, jnp.float32)),
    )
    tail_getters = [(lambda g=g: chunk(_TAIL_BASE, g))
                    for g in range(_TAIL_CH)] + [lambda: xt]
    s, mm = group_b(jnp.int32(_TAIL_BASE), tail_getters, s, mm,
                    flags_ref[jnp.int32(_MAIN_GROUPS)])

    m_ref[...] = m
    s_ref[...] = s
    mm_ref[...] = mm


def _combine_kernel(att_ref, m_ref, s_ref, mm_ref, out_ref):
    m_l = m_ref[...]
    m = jnp.max(m_l, axis=1, keepdims=True)
    s = jnp.sum(s_ref[...] * jnp.exp(m_l - m), axis=1, keepdims=True)
    mm = jnp.max(mm_ref[...], axis=1, keepdims=True)

    p = jnp.exp(att_ref[...] - m) / s
    p_mm = jnp.exp(mm - m) / s
    macro = p_mm - jnp.min(p, axis=1, keepdims=True)

    d = p[:, 1:] - p[:, :-1]
    d = ((5.0 + 5.0 * d) - 5.0) * 0.2
    w = jnp.exp(9.0 * jnp.log(1.0 + d)) - 1.0
    wm = jnp.sum(w, axis=1, keepdims=True) * (1.0 / (_K - 1))
    sl = jnp.exp(jnp.log(1.0 + wm) / 9.0) - 1.0

    cm = ((5.0 + 5.0 * macro) - 5.0) * 0.2
    cs = ((5.0 + 5.0 * sl) - 5.0) * 0.2
    v = 0.5 * (jnp.exp(10.0 * jnp.log(1.0 + cm))
               + jnp.exp(10.0 * jnp.log(1.0 + cs))) - 1.0
    out_ref[...] = jnp.exp(jnp.log(1.0 + v) / 10.0) - 1.0


def _z(*_args):
    return jnp.int32(0)


def _finish(y_pred, att32, att_vals, flags):
    grid_spec = pltpu.PrefetchScalarGridSpec(
        num_scalar_prefetch=1,
        grid=(_NRB,),
        in_specs=[
            pl.BlockSpec((_RB, _K), lambda i, flags: (i, _z())),
            pl.BlockSpec((_RB, _C), lambda i, flags: (i, _z())),
            pl.BlockSpec((_RB, _XT_W), lambda i, flags: (i, _z())),
        ],
        out_specs=[
            pl.BlockSpec((_RB, 128), lambda i, flags: (i, _z())),
            pl.BlockSpec((_RB, 128), lambda i, flags: (i, _z())),
            pl.BlockSpec((_RB, 128), lambda i, flags: (i, _z())),
        ],
    )
    m_l, s_l, mm_l = pl.pallas_call(
        _scan_kernel,
        grid_spec=grid_spec,
        out_shape=[jax.ShapeDtypeStruct((_B, 128), jnp.float32)] * 3,
        compiler_params=pltpu.CompilerParams(
            dimension_semantics=("arbitrary",),
        ),
    )(flags, att32, y_pred, y_pred[:, _XT_BASE:])

    out = pl.pallas_call(
        _combine_kernel,
        out_shape=jax.ShapeDtypeStruct((_B, 1), jnp.float32),
    )(att_vals, m_l, s_l, mm_l)
    return out.reshape(_B)


@jax.jit
def kernel(y_pred, y_attack):
    att32 = y_attack.astype(jnp.int32)
    flags = (
        jnp.zeros((_NBLK,), jnp.int32)
        .at[(att32 // _BLK).reshape(-1)]
        .set(1, mode="drop")
    )
    rows = jnp.arange(_B, dtype=jnp.int32)[:, None]
    flat_idx = (rows * _C + att32).reshape(_NW, _IDX_ROWS, 128)
    att_vals = _gather_attack(y_pred.reshape(-1), flat_idx)
    att_vals = att_vals.reshape(_B, _K)
    return _finish(y_pred, att32, att_vals, flags)
